# baseline (speedup 1.0000x reference)
import jax
import jax.numpy as jnp
from jax.experimental import pallas as pl
from jax.experimental.pallas import tpu as pltpu

N = 10000
NNZ = 320000
E = 20000
IN_CH = 128
H = 8
C = 16


def _xh_u_kernel(x_ref, w_ref, b_ref, ae_ref, xh_ref, u_ref):
    x0 = jnp.dot(x_ref[...], w_ref[...].T,
                 preferred_element_type=jnp.float32) + b_ref[...]
    xh_ref[...] = x0
    u_full = x0 * ae_ref[...]
    u_ref[...] = jnp.sum(u_full.reshape(x0.shape[0], H, C), axis=-1)


def _project(X, W_weight, W_bias, att_e):
    ae_flat = att_e.reshape(1, H * C)
    return pl.pallas_call(
        _xh_u_kernel,
        out_shape=(jax.ShapeDtypeStruct((N, H * C), jnp.float32),
                   jax.ShapeDtypeStruct((N, H), jnp.float32)),
    )(X, W_weight, W_bias.reshape(1, H * C), ae_flat)


def kernel(X, vertex, edges, V_class_index, V_class_index_aspect,
           V_class_index_user, V_class_index_item,
           W_weight, W_bias, att_v_user, att_v_item, att_v_aspect, att_e):
    Xh_flat, u = _project(X, W_weight, W_bias, att_e)
    Xh = Xh_flat.reshape(N, H, C)

    beta = jax.nn.leaky_relu(u[vertex], negative_slope=0.2)
    m = jax.ops.segment_max(beta, edges, num_segments=E)
    m = jnp.where(jnp.isfinite(m), m, 0.0)
    ex = jnp.exp(beta - m[edges])
    s = jax.ops.segment_sum(ex, edges, num_segments=E)
    betan = ex / (s[edges] + 1e-16)

    Xe_seg = jax.ops.segment_sum(Xh[vertex] * betan[..., None], edges,
                                 num_segments=E)

    t_a = (Xe_seg * att_v_aspect).sum(-1)
    t_u = (Xe_seg * att_v_user).sum(-1)
    t_i = (Xe_seg * att_v_item).sum(-1)
    T = jnp.concatenate([t_a, t_u, t_i], axis=0)
    K = jnp.concatenate([edges[V_class_index_aspect],
                         E + edges[V_class_index_user],
                         2 * E + edges[V_class_index_item]], axis=0)
    K2 = K[V_class_index]
    alpha_e = jnp.take_along_axis(T, K2, axis=0)
    alpha = jax.nn.leaky_relu(alpha_e, negative_slope=0.2)
    m2 = jax.ops.segment_max(alpha, vertex, num_segments=N)
    m2 = jnp.where(jnp.isfinite(m2), m2, 0.0)
    ex2 = jnp.exp(alpha - m2[vertex])
    s2 = jax.ops.segment_sum(ex2, vertex, num_segments=N)
    alphan = ex2 / (s2[vertex] + 1e-16)

    Xv = jax.ops.segment_sum(Xe_seg[edges] * alphan[..., None], vertex,
                             num_segments=N)
    return jax.nn.relu(Xv.reshape(N, H * C))
